# Initial kernel scaffold; baseline (speedup 1.0000x reference)
import jax
import jax.numpy as jnp
from jax.experimental import pallas as pl


def kernel(x, edge_index, batch, W1, W2, W3, W4, conv1_w, conv2_w, lin1_w, lin1_b, lin2_w, lin2_b):
    raise NotImplementedError("write your pallas kernel here")


# speedup vs baseline: 7.0894x; 7.0894x over previous
import jax
import jax.numpy as jnp
from jax import lax
from jax.experimental import pallas as pl
from jax.experimental.pallas import tpu as pltpu
from jax.experimental.pallas import tpu_sc as plsc

N = 10000
NE = 320000
HID = 32
NB = 64
KTOP = 30
NC, NS = 2, 16
NW = NC * NS
NPAD = N + 8
NACC = 10112
RPT = NACC // NS
ZROW = N
W4PAD = 8
ECHUNK = 80
EPW = NE // NW
GPW = NB // NW


def _mm1_body(x_ref, w_ref, m_ref):
    m_ref[...] = jnp.dot(x_ref[...], w_ref[...],
                         preferred_element_type=jnp.float32)


def _mm1(x, w):
    return pl.pallas_call(
        _mm1_body,
        out_shape=jax.ShapeDtypeStruct((x.shape[0], w.shape[1]), jnp.float32),
    )(x, w)


def _tanh_mm_body(p_ref, w_ref, h_ref, m_ref):
    h = jnp.tanh(p_ref[0, :N] + p_ref[1, :N])
    h_ref[...] = h
    m_ref[...] = jnp.dot(h, w_ref[...], preferred_element_type=jnp.float32)


def _tanh_mm(p, w):
    return pl.pallas_call(
        _tanh_mm_body,
        out_shape=(jax.ShapeDtypeStruct((N, p.shape[2]), jnp.float32),
                   jax.ShapeDtypeStruct((N, w.shape[1]), jnp.float32)),
    )(p, w)


def _tail_body(p_ref, batch_ref, h_ref, starts_ref, counts_ref):
    h = jnp.tanh(p_ref[0, :N, 0:1] + p_ref[1, :N, 0:1])
    h_ref[...] = h
    b = batch_ref[...]
    gids = lax.broadcasted_iota(jnp.int32, (N, NB), 1)
    onehot = (b == gids).astype(jnp.float32)
    counts_f = jnp.sum(onehot, axis=0, keepdims=True)
    i = lax.broadcasted_iota(jnp.int32, (NB, NB), 0)
    j = lax.broadcasted_iota(jnp.int32, (NB, NB), 1)
    upper = (i < j).astype(jnp.float32)
    starts_f = jnp.dot(counts_f, upper, preferred_element_type=jnp.float32)
    counts_ref[...] = counts_f.astype(jnp.int32)
    starts_ref[...] = starts_f.astype(jnp.int32)


def _tail(p, batch2):
    return pl.pallas_call(
        _tail_body,
        out_shape=(jax.ShapeDtypeStruct((N, 1), jnp.float32),
                   jax.ShapeDtypeStruct((1, NB), jnp.int32),
                   jax.ShapeDtypeStruct((1, NB), jnp.int32)),
    )(p, batch2)


def _readout_body(d_ref, bw_ref, s_ref, dd_ref, m2_ref,
                  l1w_ref, l1b_ref, l2w_ref, l2b_ref, o_ref):
    z1 = jnp.maximum(
        jnp.dot(d_ref[...], bw_ref[...], preferred_element_type=jnp.float32),
        0.0)
    zp = 0.5 * (jnp.dot(z1, s_ref[...], preferred_element_type=jnp.float32)
                + jnp.abs(jnp.dot(z1, dd_ref[...],
                                  preferred_element_type=jnp.float32)))
    z2 = jnp.maximum(
        jnp.dot(zp, m2_ref[...], preferred_element_type=jnp.float32), 0.0)
    z3 = jnp.maximum(
        jnp.dot(z2, l1w_ref[...], preferred_element_type=jnp.float32)
        + l1b_ref[...], 0.0)
    o_ref[...] = (jnp.dot(z3, l2w_ref[...], preferred_element_type=jnp.float32)
                  + l2b_ref[...])


def _readout(dense2, bw, s, dd, m2, l1w, l1b, l2w, l2b):
    return pl.pallas_call(
        _readout_body,
        out_shape=jax.ShapeDtypeStruct((NB, 1), jnp.float32),
    )(dense2, bw, s, dd, m2, l1w, l1b, l2w, l2b)


def _sc_scatter_body(m_hbm, src_hbm, dst_hbm, zeros_hbm, out_hbm,
                     acc_sh, src_v, dst_v, rows_v, sem):
    cid = lax.axis_index("c")
    sid = lax.axis_index("s")
    wid = sid * NC + cid
    r0 = sid * RPT
    pltpu.sync_copy(zeros_hbm.at[pl.ds(r0, RPT)], acc_sh.at[pl.ds(r0, RPT)])
    plsc.subcore_barrier()
    base = wid * EPW

    def body(jj, carry):
        off = base + jj * ECHUNK
        pltpu.sync_copy(src_hbm.at[pl.ds(off, ECHUNK)], src_v)
        pltpu.sync_copy(dst_hbm.at[pl.ds(off, ECHUNK)], dst_v)
        pltpu.async_copy(m_hbm.at[src_v], rows_v, sem).wait()
        pltpu.sync_copy(rows_v, acc_sh.at[dst_v], add=True)
        return carry

    lax.fori_loop(0, EPW // ECHUNK, body, jnp.int32(0))
    plsc.subcore_barrier()
    pltpu.sync_copy(acc_sh.at[pl.ds(r0, RPT)], out_hbm.at[cid, pl.ds(r0, RPT)])


def _sc_scatter(m, src, dst, width):
    zeros = jnp.zeros((NACC, width), jnp.float32)
    f = pl.kernel(
        _sc_scatter_body,
        out_type=jax.ShapeDtypeStruct((NC, NACC, width), jnp.float32),
        mesh=plsc.VectorSubcoreMesh(core_axis_name="c", subcore_axis_name="s",
                                    num_cores=NC, num_subcores=NS),
        compiler_params=pltpu.CompilerParams(use_tc_tiling_on_sc=False),
        scratch_types=[
            pltpu.VMEM_SHARED((NACC, width), jnp.float32),
            pltpu.VMEM((ECHUNK,), jnp.int32),
            pltpu.VMEM((ECHUNK,), jnp.int32),
            pltpu.VMEM((ECHUNK, width), jnp.float32),
            pltpu.SemaphoreType.DMA,
        ],
    )
    return f(m, src, dst, zeros)


def _sc_topk_body(keys_hbm, starts_hbm, counts_hbm, xcat_hbm, dense_hbm,
                  keys_v, starts_v, counts_v, idx_v, rows_v, sem):
    cid = lax.axis_index("c")
    sid = lax.axis_index("s")
    wid = sid * NC + cid
    pltpu.sync_copy(keys_hbm, keys_v)
    pltpu.sync_copy(starts_hbm, starts_v)
    pltpu.sync_copy(counts_hbm, counts_v)
    neg = jnp.float32(-3.0e38)
    bigi = jnp.int32(1 << 30)
    lane = lax.iota(jnp.int32, 16)
    for gg in range(GPW):
        g = wid * GPW + gg
        s0 = jnp.int32(0)
        cnt = jnp.int32(0)
        for cc in range(NB // 16):
            sv = starts_v[pl.ds(cc * 16, 16)]
            cv = counts_v[pl.ds(cc * 16, 16)]
            hit = (cc * 16 + lane) == g
            s0 = s0 + jnp.sum(jnp.where(hit, sv, 0))
            cnt = cnt + jnp.sum(jnp.where(hit, cv, 0))
        base0 = (s0 // 16) * 16
        end = s0 + cnt
        nch = (end - base0 + 15) // 16
        sel0 = jnp.full((16,), ZROW, jnp.int32)
        sel1 = jnp.full((16,), ZROW, jnp.int32)
        for s in range(KTOP):
            def chunk_body(jc, carry):
                mv, mi = carry
                off = base0 + jc * 16
                v = keys_v[pl.ds(off, 16)]
                posa = off + lane
                valid = (posa >= s0) & (posa < end)
                v = jnp.where(valid, v, neg)
                upd = v > mv
                return jnp.where(upd, v, mv), jnp.where(upd, posa, mi)

            mv, mi = lax.fori_loop(
                0, nch, chunk_body,
                (jnp.full((16,), neg, jnp.float32),
                 jnp.full((16,), bigi, jnp.int32)))
            mtop = jnp.max(mv)
            cand = jnp.where(mv == mtop, mi, bigi)
            r = jnp.min(cand)
            have = jnp.int32(s) < cnt
            selv = jnp.where(have, r, ZROW)
            if s < 16:
                sel0 = jnp.where(lane == s, selv, sel0)
            else:
                sel1 = jnp.where(lane == (s - 16), selv, sel1)
            plsc.store_scatter(keys_v, [jnp.full((16,), r, jnp.int32)],
                               jnp.full((16,), neg, jnp.float32),
                               mask=(lane == 0) & have)
        idx_v[pl.ds(0, 16)] = sel0
        idx_v[pl.ds(16, 16)] = sel1
        pltpu.async_copy(xcat_hbm.at[idx_v], rows_v, sem).wait()
        pltpu.sync_copy(rows_v, dense_hbm.at[g])


def _sc_topk(keys, starts, counts, xcat_pad):
    f = pl.kernel(
        _sc_topk_body,
        out_type=jax.ShapeDtypeStruct((NB, 32, 128), jnp.float32),
        mesh=plsc.VectorSubcoreMesh(core_axis_name="c", subcore_axis_name="s",
                                    num_cores=NC, num_subcores=NS),
        compiler_params=pltpu.CompilerParams(needs_layout_passes=False),
        scratch_types=[
            pltpu.VMEM((N,), jnp.float32),
            pltpu.VMEM((NB,), jnp.int32),
            pltpu.VMEM((NB,), jnp.int32),
            pltpu.VMEM((32,), jnp.int32),
            pltpu.VMEM((32, 128), jnp.float32),
            pltpu.SemaphoreType.DMA,
        ],
    )
    return f(keys, starts, counts, xcat_pad)


def kernel(x, edge_index, batch, W1, W2, W3, W4, conv1_w, conv2_w,
           lin1_w, lin1_b, lin2_w, lin2_b):
    src = edge_index[0]
    dst = edge_index[1]

    m1 = _mm1(x, W1)
    p1 = _sc_scatter(m1, src, dst, HID)
    h1, m2 = _tanh_mm(p1, W2)
    p2 = _sc_scatter(m2, src, dst, HID)
    h2, m3 = _tanh_mm(p2, W3)
    p3 = _sc_scatter(m3, src, dst, HID)
    w4p = jnp.zeros((HID, W4PAD), jnp.float32).at[:, 0:1].set(W4)
    h3, m4 = _tanh_mm(p3, w4p)
    p4 = _sc_scatter(m4, src, dst, W4PAD)

    batch2 = batch.reshape(N, 1).astype(jnp.int32)
    h4, starts2, counts2 = _tail(p4, batch2)

    xcat = jnp.concatenate([h1, h2, h3, h4], axis=1)
    xcat_pad = jnp.zeros((NPAD, 128), jnp.float32).at[:N, :97].set(xcat)
    keys = h4.reshape(N)
    dense = _sc_topk(keys, starts2.reshape(NB), counts2.reshape(NB),
                     xcat_pad)
    dense2 = dense.reshape(NB, 32 * 128)

    w1c = jnp.zeros((128, 4), jnp.float32).at[:97].set(conv1_w[:, 0, :].T)
    eye = jnp.eye(32, dtype=jnp.float32)[:, :30]
    big_w1 = (eye[:, None, :, None] * w1c[None, :, None, :]).reshape(4096, 120)
    ar120 = jnp.arange(120)
    t_i, o_i = ar120 // 4, ar120 % 4
    ar60 = jnp.arange(60)
    zt, zo = ar60 // 4, ar60 % 4
    s_mat = ((o_i[:, None] == zo[None, :])
             & ((t_i[:, None] // 2) == zt[None, :])).astype(jnp.float32)
    d_mat = s_mat * (1.0 - 2.0 * (t_i % 2).astype(jnp.float32))[:, None]
    ar88 = jnp.arange(88)
    o2c, t3c = ar88 // 11, ar88 % 11
    rdiff = zt[:, None] - t3c[None, :]
    m2_mat = jnp.where((rdiff >= 0) & (rdiff <= 4),
                       conv2_w[o2c[None, :], zo[:, None],
                               jnp.clip(rdiff, 0, 4)],
                       0.0).astype(jnp.float32)

    return _readout(dense2, big_w1, s_mat, d_mat, m2_mat,
                    lin1_w.T, lin1_b.reshape(1, 10),
                    lin2_w.T, lin2_b.reshape(1, 1))


# speedup vs baseline: 11.4019x; 1.6083x over previous
import jax
import jax.numpy as jnp
from jax import lax
from jax.experimental import pallas as pl
from jax.experimental.pallas import tpu as pltpu
from jax.experimental.pallas import tpu_sc as plsc

N = 10000
NE = 320000
HID = 32
NB = 64
KTOP = 30
NC, NS = 2, 16
NW = NC * NS
NPAD = N + 8
NACC = 10112
RPT = NACC // NS
ZROW = N
W4PAD = 8
ECHUNK = 128
NCHUNK = 80
EPW = NCHUNK * ECHUNK
NEP = EPW * NW
EDUMMY = 10100
NBUF = 4
GPW = NB // NW


def _mm1_body(x_ref, w_ref, m_ref):
    m_ref[...] = jnp.dot(x_ref[...], w_ref[...],
                         preferred_element_type=jnp.float32)


def _mm1(x, w):
    return pl.pallas_call(
        _mm1_body,
        out_shape=jax.ShapeDtypeStruct((x.shape[0], w.shape[1]), jnp.float32),
    )(x, w)


def _tanh_mm_body(p_ref, w_ref, h_ref, m_ref):
    h = jnp.tanh(p_ref[0, :N] + p_ref[1, :N])
    h_ref[...] = h
    m_ref[...] = jnp.dot(h, w_ref[...], preferred_element_type=jnp.float32)


def _tanh_mm(p, w):
    return pl.pallas_call(
        _tanh_mm_body,
        out_shape=(jax.ShapeDtypeStruct((N, p.shape[2]), jnp.float32),
                   jax.ShapeDtypeStruct((N, w.shape[1]), jnp.float32)),
    )(p, w)


def _tail_body(p_ref, batch_ref, h_ref, starts_ref, counts_ref):
    h = jnp.tanh(p_ref[0, :N, 0:1] + p_ref[1, :N, 0:1])
    h_ref[...] = h
    b = batch_ref[...]
    gids = lax.broadcasted_iota(jnp.int32, (N, NB), 1)
    onehot = (b == gids).astype(jnp.float32)
    counts_f = jnp.sum(onehot, axis=0, keepdims=True)
    i = lax.broadcasted_iota(jnp.int32, (NB, NB), 0)
    j = lax.broadcasted_iota(jnp.int32, (NB, NB), 1)
    upper = (i < j).astype(jnp.float32)
    starts_f = jnp.dot(counts_f, upper, preferred_element_type=jnp.float32)
    counts_ref[...] = counts_f.astype(jnp.int32)
    starts_ref[...] = starts_f.astype(jnp.int32)


def _tail(p, batch2):
    return pl.pallas_call(
        _tail_body,
        out_shape=(jax.ShapeDtypeStruct((N, 1), jnp.float32),
                   jax.ShapeDtypeStruct((1, NB), jnp.int32),
                   jax.ShapeDtypeStruct((1, NB), jnp.int32)),
    )(p, batch2)


def _readout_body(d_ref, bw_ref, s_ref, dd_ref, m2_ref,
                  l1w_ref, l1b_ref, l2w_ref, l2b_ref, o_ref):
    z1 = jnp.maximum(
        jnp.dot(d_ref[...], bw_ref[...], preferred_element_type=jnp.float32),
        0.0)
    zp = 0.5 * (jnp.dot(z1, s_ref[...], preferred_element_type=jnp.float32)
                + jnp.abs(jnp.dot(z1, dd_ref[...],
                                  preferred_element_type=jnp.float32)))
    z2 = jnp.maximum(
        jnp.dot(zp, m2_ref[...], preferred_element_type=jnp.float32), 0.0)
    z3 = jnp.maximum(
        jnp.dot(z2, l1w_ref[...], preferred_element_type=jnp.float32)
        + l1b_ref[...], 0.0)
    o_ref[...] = (jnp.dot(z3, l2w_ref[...], preferred_element_type=jnp.float32)
                  + l2b_ref[...])


def _readout(dense2, bw, s, dd, m2, l1w, l1b, l2w, l2b):
    return pl.pallas_call(
        _readout_body,
        out_shape=jax.ShapeDtypeStruct((NB, 1), jnp.float32),
    )(dense2, bw, s, dd, m2, l1w, l1b, l2w, l2b)


def _sc_scatter_body(m_hbm, src_hbm, dst_hbm, zeros_hbm, out_hbm,
                     acc_sh, src_v, dst_v, rows0, rows1, rows2, rows3,
                     gs0, gs1, gs2, gs3, ss0, ss1, ss2, ss3):
    rows = (rows0, rows1, rows2, rows3)
    gsem = (gs0, gs1, gs2, gs3)
    ssem = (ss0, ss1, ss2, ss3)
    cid = lax.axis_index("c")
    sid = lax.axis_index("s")
    wid = sid * NC + cid
    r0 = sid * RPT

    pltpu.sync_copy(src_hbm.at[wid], src_v)
    pltpu.sync_copy(dst_hbm.at[wid], dst_v)

    def start_gather(c, b):
        return pltpu.async_copy(m_hbm.at[src_v.at[c]], rows[b], gsem[b])

    def wait_gather(c, b):
        pltpu.make_async_copy(m_hbm.at[src_v.at[c]], rows[b], gsem[b]).wait()

    def start_scatter(c, b):
        return pltpu.async_copy(rows[b], acc_sh.at[dst_v.at[c]], ssem[b],
                                add=True)

    def wait_scatter(c, b):
        pltpu.make_async_copy(rows[b], acc_sh.at[dst_v.at[c]],
                              ssem[b]).wait()

    for b in range(NBUF):
        start_gather(b, b)

    pltpu.sync_copy(zeros_hbm.at[pl.ds(r0, RPT)], acc_sh.at[pl.ds(r0, RPT)])
    plsc.subcore_barrier()

    def group(j, carry):
        for b in range(NBUF):
            c = j * NBUF + b
            wait_gather(c, b)
            start_scatter(c, b)
        for b in range(NBUF):
            c = j * NBUF + b
            wait_scatter(c, b)
            start_gather(c + NBUF, b)
        return carry

    lax.fori_loop(0, NCHUNK // NBUF - 1, group, jnp.int32(0))
    for b in range(NBUF):
        c = NCHUNK - NBUF + b
        wait_gather(c, b)
        start_scatter(c, b)
    for b in range(NBUF):
        c = NCHUNK - NBUF + b
        wait_scatter(c, b)

    plsc.subcore_barrier()
    pltpu.sync_copy(acc_sh.at[pl.ds(r0, RPT)], out_hbm.at[cid, pl.ds(r0, RPT)])


def _sc_scatter(m, src3, dst3, width):
    zeros = jnp.zeros((NACC, width), jnp.float32)
    f = pl.kernel(
        _sc_scatter_body,
        out_type=jax.ShapeDtypeStruct((NC, NACC, width), jnp.float32),
        mesh=plsc.VectorSubcoreMesh(core_axis_name="c", subcore_axis_name="s",
                                    num_cores=NC, num_subcores=NS),
        compiler_params=pltpu.CompilerParams(use_tc_tiling_on_sc=False),
        scratch_types=(
            [pltpu.VMEM_SHARED((NACC, width), jnp.float32),
             pltpu.VMEM((NCHUNK, ECHUNK), jnp.int32),
             pltpu.VMEM((NCHUNK, ECHUNK), jnp.int32)]
            + [pltpu.VMEM((ECHUNK, width), jnp.float32)] * NBUF
            + [pltpu.SemaphoreType.DMA] * (2 * NBUF)
        ),
    )
    return f(m, src3, dst3, zeros)


def _sc_topk_body(keys_hbm, starts_hbm, counts_hbm, xcat_hbm, dense_hbm,
                  keys_v, starts_v, counts_v, idx_v, rows_v, sem):
    cid = lax.axis_index("c")
    sid = lax.axis_index("s")
    wid = sid * NC + cid
    pltpu.sync_copy(keys_hbm, keys_v)
    pltpu.sync_copy(starts_hbm, starts_v)
    pltpu.sync_copy(counts_hbm, counts_v)
    neg = jnp.float32(-3.0e38)
    bigi = jnp.int32(1 << 30)
    lane = lax.iota(jnp.int32, 16)
    for gg in range(GPW):
        g = wid * GPW + gg
        s0 = jnp.int32(0)
        cnt = jnp.int32(0)
        for cc in range(NB // 16):
            sv = starts_v[pl.ds(cc * 16, 16)]
            cv = counts_v[pl.ds(cc * 16, 16)]
            hit = (cc * 16 + lane) == g
            s0 = s0 + jnp.sum(jnp.where(hit, sv, 0))
            cnt = cnt + jnp.sum(jnp.where(hit, cv, 0))
        base0 = (s0 // 16) * 16
        end = s0 + cnt
        nch = (end - base0 + 15) // 16
        sel0 = jnp.full((16,), ZROW, jnp.int32)
        sel1 = jnp.full((16,), ZROW, jnp.int32)
        for s in range(KTOP):
            def chunk_body(jc, carry):
                mv, mi = carry
                off = base0 + jc * 16
                v = keys_v[pl.ds(off, 16)]
                posa = off + lane
                valid = (posa >= s0) & (posa < end)
                v = jnp.where(valid, v, neg)
                upd = v > mv
                return jnp.where(upd, v, mv), jnp.where(upd, posa, mi)

            mv, mi = lax.fori_loop(
                0, nch, chunk_body,
                (jnp.full((16,), neg, jnp.float32),
                 jnp.full((16,), bigi, jnp.int32)))
            mtop = jnp.max(mv)
            cand = jnp.where(mv == mtop, mi, bigi)
            r = jnp.min(cand)
            have = jnp.int32(s) < cnt
            selv = jnp.where(have, r, ZROW)
            if s < 16:
                sel0 = jnp.where(lane == s, selv, sel0)
            else:
                sel1 = jnp.where(lane == (s - 16), selv, sel1)
            plsc.store_scatter(keys_v, [jnp.full((16,), r, jnp.int32)],
                               jnp.full((16,), neg, jnp.float32),
                               mask=(lane == 0) & have)
        idx_v[pl.ds(0, 16)] = sel0
        idx_v[pl.ds(16, 16)] = sel1
        pltpu.async_copy(xcat_hbm.at[idx_v], rows_v, sem).wait()
        pltpu.sync_copy(rows_v, dense_hbm.at[g])


def _sc_topk(keys, starts, counts, xcat_pad):
    f = pl.kernel(
        _sc_topk_body,
        out_type=jax.ShapeDtypeStruct((NB, 32, 128), jnp.float32),
        mesh=plsc.VectorSubcoreMesh(core_axis_name="c", subcore_axis_name="s",
                                    num_cores=NC, num_subcores=NS),
        compiler_params=pltpu.CompilerParams(needs_layout_passes=False),
        scratch_types=[
            pltpu.VMEM((N,), jnp.float32),
            pltpu.VMEM((NB,), jnp.int32),
            pltpu.VMEM((NB,), jnp.int32),
            pltpu.VMEM((32,), jnp.int32),
            pltpu.VMEM((32, 128), jnp.float32),
            pltpu.SemaphoreType.DMA,
        ],
    )
    return f(keys, starts, counts, xcat_pad)


def kernel(x, edge_index, batch, W1, W2, W3, W4, conv1_w, conv2_w,
           lin1_w, lin1_b, lin2_w, lin2_b):
    pad = NEP - NE
    src3 = jnp.concatenate(
        [edge_index[0], jnp.zeros((pad,), edge_index.dtype)]
    ).astype(jnp.int32).reshape(NW, NCHUNK, ECHUNK)
    dst3 = jnp.concatenate(
        [edge_index[1], jnp.full((pad,), EDUMMY, edge_index.dtype)]
    ).astype(jnp.int32).reshape(NW, NCHUNK, ECHUNK)

    m1 = _mm1(x, W1)
    p1 = _sc_scatter(m1, src3, dst3, HID)
    h1, m2 = _tanh_mm(p1, W2)
    p2 = _sc_scatter(m2, src3, dst3, HID)
    h2, m3 = _tanh_mm(p2, W3)
    p3 = _sc_scatter(m3, src3, dst3, HID)
    w4p = jnp.zeros((HID, W4PAD), jnp.float32).at[:, 0:1].set(W4)
    h3, m4 = _tanh_mm(p3, w4p)
    p4 = _sc_scatter(m4, src3, dst3, W4PAD)

    batch2 = batch.reshape(N, 1).astype(jnp.int32)
    h4, starts2, counts2 = _tail(p4, batch2)

    xcat = jnp.concatenate([h1, h2, h3, h4], axis=1)
    xcat_pad = jnp.zeros((NPAD, 128), jnp.float32).at[:N, :97].set(xcat)
    keys = h4.reshape(N)
    dense = _sc_topk(keys, starts2.reshape(NB), counts2.reshape(NB),
                     xcat_pad)
    dense2 = dense.reshape(NB, 32 * 128)

    w1c = jnp.zeros((128, 4), jnp.float32).at[:97].set(conv1_w[:, 0, :].T)
    eye = jnp.eye(32, dtype=jnp.float32)[:, :30]
    big_w1 = (eye[:, None, :, None] * w1c[None, :, None, :]).reshape(4096, 120)
    ar120 = jnp.arange(120)
    t_i, o_i = ar120 // 4, ar120 % 4
    ar60 = jnp.arange(60)
    zt, zo = ar60 // 4, ar60 % 4
    s_mat = ((o_i[:, None] == zo[None, :])
             & ((t_i[:, None] // 2) == zt[None, :])).astype(jnp.float32)
    d_mat = s_mat * (1.0 - 2.0 * (t_i % 2).astype(jnp.float32))[:, None]
    ar88 = jnp.arange(88)
    o2c, t3c = ar88 // 11, ar88 % 11
    rdiff = zt[:, None] - t3c[None, :]
    m2_mat = jnp.where((rdiff >= 0) & (rdiff <= 4),
                       conv2_w[o2c[None, :], zo[:, None],
                               jnp.clip(rdiff, 0, 4)],
                       0.0).astype(jnp.float32)

    return _readout(dense2, big_w1, s_mat, d_mat, m2_mat,
                    lin1_w.T, lin1_b.reshape(1, 10),
                    lin2_w.T, lin2_b.reshape(1, 1))
